# baseline (speedup 1.0000x reference)
import functools
import math

import jax
import jax.numpy as jnp
from jax import lax
from jax.experimental import pallas as pl
from jax.experimental.pallas import tpu as pltpu
from jax.experimental.pallas import tpu_sc as plsc

NC = 2
NS = 16
L = 16
NW = NC * NS

VOCAB = 1000
D = 128
NUM_TABLES = 3
SCALE = math.sqrt(128.0)

BBLK = 128
STAGE_ROWS = 32


def _body(ph_t, onset, rhyme, tone, out, sh0, sh1, sh2,
          ix00, ix01, ix10, ix11, ix20, ix21,
          r00, r01, r10, r11, r20, r21, *sems):
    cid = lax.axis_index("c")
    sid = lax.axis_index("s")
    wid = sid * NC + cid

    seq = out.shape[0]
    b0 = wid * BBLK
    shared = (sh0, sh1, sh2)
    idx = ((ix00, ix01), (ix10, ix11), (ix20, ix21))
    rows = ((r00, r01), (r10, r11), (r20, r21))
    semg = (sems[0:2], sems[2:4], sems[4:6])
    semw = (sems[6:8], sems[8:10], sems[10:12])
    semi = (sems[12:14], sems[14:16], sems[16:18])

    def fire_idx(j, p, s):
        pltpu.async_copy(ph_t.at[j, s, pl.ds(b0, BBLK)], idx[j][p],
                         semi[j][p])

    def wait_idx(j, p, s):
        pltpu.make_async_copy(ph_t.at[j, s, pl.ds(b0, BBLK)], idx[j][p],
                              semi[j][p]).wait()

    def fire_gather(j, p):
        pltpu.async_copy(shared[j].at[idx[j][p]], rows[j][p], semg[j][p])

    def wait_gather(j, p):
        pltpu.make_async_copy(shared[j].at[idx[j][p]], rows[j][p],
                              semg[j][p]).wait()

    def out_slice(j, s):
        return out.at[s, pl.ds(b0, BBLK), pl.ds(j * D, D)]

    def fire_write(j, p, s):
        pltpu.async_copy(rows[j][p], out_slice(j, s), semw[j][p])

    def wait_write(j, p, s):
        pltpu.make_async_copy(rows[j][p], out_slice(j, s),
                              semw[j][p]).wait()

    for p in range(2):
        for j in range(NUM_TABLES):
            fire_idx(j, p, jnp.int32(p))

    for j, tab in enumerate((onset, rhyme, tone)):
        for k in range(2):
            t0 = jnp.minimum(sid * (2 * STAGE_ROWS), VOCAB - 2 * STAGE_ROWS) \
                + k * STAGE_ROWS
            pltpu.sync_copy(tab.at[pl.ds(t0, STAGE_ROWS)],
                            r00.at[pl.ds(0, STAGE_ROWS)])

            def scale_row(i, _):
                for h in range(D // L):
                    r00[i, pl.ds(h * L, L)] = r00[i, pl.ds(h * L, L)] * SCALE
                return 0

            lax.fori_loop(0, STAGE_ROWS, scale_row, 0)
            pltpu.sync_copy(r00.at[pl.ds(0, STAGE_ROWS)],
                            shared[j].at[pl.ds(t0, STAGE_ROWS)])
    plsc.subcore_barrier()

    for p in range(2):
        for j in range(NUM_TABLES):
            wait_idx(j, p, jnp.int32(p))
            fire_gather(j, p)

    def pair_body(i, _):
        sa = 2 * i
        sb = 2 * i + 1
        sa2 = jnp.where(sa + 2 < seq, sa + 2, 0)
        sb2 = jnp.where(sb + 2 < seq, sb + 2, 0)
        for p, s, s2 in ((0, sa, sa2), (1, sb, sb2)):
            for j in range(NUM_TABLES):
                wait_gather(j, p)
            for j in range(NUM_TABLES):
                fire_write(j, p, s)
            for j in range(NUM_TABLES):
                fire_idx(j, p, s2)
        for p, s, s2 in ((0, sa, sa2), (1, sb, sb2)):
            for j in range(NUM_TABLES):
                wait_write(j, p, s)
                wait_idx(j, p, s2)
                fire_gather(j, p)
        return 0

    lax.fori_loop(0, seq // 2, pair_body, 0)
    for p in range(2):
        for j in range(NUM_TABLES):
            wait_gather(j, p)


def _make(n_batch, seq):
    mesh = plsc.VectorSubcoreMesh(core_axis_name="c", subcore_axis_name="s")
    return pl.kernel(
        _body,
        out_type=jax.ShapeDtypeStruct((seq, n_batch, NUM_TABLES * D),
                                      jnp.float32),
        mesh=mesh,
        compiler_params=pltpu.CompilerParams(needs_layout_passes=False),
        scratch_types=[
            pltpu.VMEM_SHARED((VOCAB, D), jnp.float32),
            pltpu.VMEM_SHARED((VOCAB, D), jnp.float32),
            pltpu.VMEM_SHARED((VOCAB, D), jnp.float32),
            pltpu.VMEM((BBLK,), jnp.int32),
            pltpu.VMEM((BBLK,), jnp.int32),
            pltpu.VMEM((BBLK,), jnp.int32),
            pltpu.VMEM((BBLK,), jnp.int32),
            pltpu.VMEM((BBLK,), jnp.int32),
            pltpu.VMEM((BBLK,), jnp.int32),
            pltpu.VMEM((BBLK, D), jnp.float32),
            pltpu.VMEM((BBLK, D), jnp.float32),
            pltpu.VMEM((BBLK, D), jnp.float32),
            pltpu.VMEM((BBLK, D), jnp.float32),
            pltpu.VMEM((BBLK, D), jnp.float32),
            pltpu.VMEM((BBLK, D), jnp.float32),
        ] + [pltpu.SemaphoreType.DMA] * 18,
    )


@jax.jit
def kernel(phoneme_tensor, onset_table, rhyme_table, tone_table):
    b, s, _ = phoneme_tensor.shape
    ph_t = phoneme_tensor.astype(jnp.int32).transpose(2, 1, 0)
    out3 = _make(b, s)(ph_t, onset_table, rhyme_table, tone_table)
    return out3.transpose(1, 0, 2)


# speedup vs baseline: 1.0003x; 1.0003x over previous
import functools
import math

import jax
import jax.numpy as jnp
from jax import lax
from jax.experimental import pallas as pl
from jax.experimental.pallas import tpu as pltpu
from jax.experimental.pallas import tpu_sc as plsc

NC = 2
NS = 16
L = 16
NW = NC * NS

VOCAB = 1000
D = 128
NUM_TABLES = 3
SCALE = math.sqrt(128.0)

BBLK = 128
STAGE_ROWS = 32


def _body(ph_t, onset, rhyme, tone, out, sh0, sh1, sh2,
          ib0, ib1, r00, r01, r10, r11, r20, r21, *sems):
    cid = lax.axis_index("c")
    sid = lax.axis_index("s")
    wid = sid * NC + cid

    seq = out.shape[0]
    b0 = wid * BBLK
    shared = (sh0, sh1, sh2)
    ibuf = (ib0, ib1)
    rows = ((r00, r01), (r10, r11), (r20, r21))
    semg = (sems[0:2], sems[2:4], sems[4:6])
    semw = (sems[6:8], sems[8:10], sems[10:12])
    semi = sems[12:14]

    def fire_idx(p, s):
        pltpu.async_copy(ph_t.at[:, s, pl.ds(b0, BBLK)], ibuf[p], semi[p])

    def wait_idx(p, s):
        pltpu.make_async_copy(ph_t.at[:, s, pl.ds(b0, BBLK)], ibuf[p],
                              semi[p]).wait()

    def fire_gather(j, p):
        pltpu.async_copy(shared[j].at[ibuf[p].at[j]], rows[j][p], semg[j][p])

    def wait_gather(j, p):
        pltpu.make_async_copy(shared[j].at[ibuf[p].at[j]], rows[j][p],
                              semg[j][p]).wait()

    def out_slice(j, s):
        return out.at[s, pl.ds(b0, BBLK), pl.ds(j * D, D)]

    def fire_write(j, p, s):
        pltpu.async_copy(rows[j][p], out_slice(j, s), semw[j][p])

    def wait_write(j, p, s):
        pltpu.make_async_copy(rows[j][p], out_slice(j, s),
                              semw[j][p]).wait()

    for p in range(2):
        fire_idx(p, jnp.int32(p))

    for j, tab in enumerate((onset, rhyme, tone)):
        for k in range(2):
            t0 = jnp.minimum(sid * (2 * STAGE_ROWS), VOCAB - 2 * STAGE_ROWS) \
                + k * STAGE_ROWS
            pltpu.sync_copy(tab.at[pl.ds(t0, STAGE_ROWS)],
                            r00.at[pl.ds(0, STAGE_ROWS)])

            def scale_row(i, _):
                for h in range(D // L):
                    r00[i, pl.ds(h * L, L)] = r00[i, pl.ds(h * L, L)] * SCALE
                return 0

            lax.fori_loop(0, STAGE_ROWS, scale_row, 0)
            pltpu.sync_copy(r00.at[pl.ds(0, STAGE_ROWS)],
                            shared[j].at[pl.ds(t0, STAGE_ROWS)])
    plsc.subcore_barrier()

    for p in range(2):
        wait_idx(p, jnp.int32(p))
        for j in range(NUM_TABLES):
            fire_gather(j, p)

    def pair_body(i, _):
        sa = 2 * i
        sb = 2 * i + 1
        sa2 = jnp.where(sa + 2 < seq, sa + 2, 0)
        sb2 = jnp.where(sb + 2 < seq, sb + 2, 0)
        for p, s, s2 in ((0, sa, sa2), (1, sb, sb2)):
            for j in range(NUM_TABLES):
                wait_gather(j, p)
            for j in range(NUM_TABLES):
                fire_write(j, p, s)
            fire_idx(p, s2)
        for p, s, s2 in ((0, sa, sa2), (1, sb, sb2)):
            wait_idx(p, s2)
            for j in range(NUM_TABLES):
                wait_write(j, p, s)
                fire_gather(j, p)
        return 0

    lax.fori_loop(0, seq // 2, pair_body, 0)
    for p in range(2):
        for j in range(NUM_TABLES):
            wait_gather(j, p)


def _make(n_batch, seq):
    mesh = plsc.VectorSubcoreMesh(core_axis_name="c", subcore_axis_name="s")
    return pl.kernel(
        _body,
        out_type=jax.ShapeDtypeStruct((seq, n_batch, NUM_TABLES * D),
                                      jnp.float32),
        mesh=mesh,
        compiler_params=pltpu.CompilerParams(needs_layout_passes=False),
        scratch_types=[
            pltpu.VMEM_SHARED((VOCAB, D), jnp.float32),
            pltpu.VMEM_SHARED((VOCAB, D), jnp.float32),
            pltpu.VMEM_SHARED((VOCAB, D), jnp.float32),
            pltpu.VMEM((NUM_TABLES, BBLK), jnp.int32),
            pltpu.VMEM((NUM_TABLES, BBLK), jnp.int32),
            pltpu.VMEM((BBLK, D), jnp.float32),
            pltpu.VMEM((BBLK, D), jnp.float32),
            pltpu.VMEM((BBLK, D), jnp.float32),
            pltpu.VMEM((BBLK, D), jnp.float32),
            pltpu.VMEM((BBLK, D), jnp.float32),
            pltpu.VMEM((BBLK, D), jnp.float32),
        ] + [pltpu.SemaphoreType.DMA] * 14,
    )


@jax.jit
def kernel(phoneme_tensor, onset_table, rhyme_table, tone_table):
    b, s, _ = phoneme_tensor.shape
    ph_t = phoneme_tensor.astype(jnp.int32).transpose(2, 1, 0)
    out3 = _make(b, s)(ph_t, onset_table, rhyme_table, tone_table)
    return out3.transpose(1, 0, 2)


# speedup vs baseline: 1.0440x; 1.0437x over previous
import functools
import math

import jax
import jax.numpy as jnp
from jax import lax
from jax.experimental import pallas as pl
from jax.experimental.pallas import tpu as pltpu
from jax.experimental.pallas import tpu_sc as plsc

NC = 2
NS = 16
L = 16
NW = NC * NS

VOCAB = 1000
D = 128
NUM_TABLES = 3
SCALE = math.sqrt(128.0)

BBLK = 128
CB = 64
NP = 4
STAGE_ROWS = 32


def _body(ph_t, onset, rhyme, tone, out, sh0, sh1, sh2, stg, *rest):
    cid = lax.axis_index("c")
    sid = lax.axis_index("s")
    wid = sid * NC + cid

    seq = out.shape[0]
    nch = 2 * seq
    b0 = wid * BBLK
    shared = (sh0, sh1, sh2)
    ibuf = rest[0:NP]
    rows = tuple(tuple(rest[NP + j * NP + p] for p in range(NP))
                 for j in range(NUM_TABLES))
    sems = rest[NP + NUM_TABLES * NP:]
    semg = tuple(sems[j * NP:(j + 1) * NP] for j in range(NUM_TABLES))
    semw = tuple(sems[(NUM_TABLES + j) * NP:(NUM_TABLES + j + 1) * NP]
                 for j in range(NUM_TABLES))
    semi = sems[2 * NUM_TABLES * NP:]

    def chunk_pos(c):
        return c // 2, b0 + (c % 2) * CB

    def fire_idx(p, c):
        s, _ = chunk_pos(c)
        pltpu.async_copy(ph_t.at[:, s, pl.ds(b0, BBLK)], ibuf[p], semi[p])

    def wait_idx(p, c):
        s, _ = chunk_pos(c)
        pltpu.make_async_copy(ph_t.at[:, s, pl.ds(b0, BBLK)], ibuf[p],
                              semi[p]).wait()

    def idx_run(j, p, c):
        return ibuf[p].at[j, pl.ds((c % 2) * CB, CB)]

    def fire_gather(j, p, c):
        pltpu.async_copy(shared[j].at[idx_run(j, p, c)], rows[j][p],
                         semg[j][p])

    def wait_gather(j, p, c):
        pltpu.make_async_copy(shared[j].at[idx_run(j, p, c)], rows[j][p],
                              semg[j][p]).wait()

    def out_slice(j, c):
        s, bc = chunk_pos(c)
        return out.at[s, pl.ds(bc, CB), pl.ds(j * D, D)]

    def fire_write(j, p, c):
        pltpu.async_copy(rows[j][p], out_slice(j, c), semw[j][p])

    def wait_write(j, p, c):
        pltpu.make_async_copy(rows[j][p], out_slice(j, c),
                              semw[j][p]).wait()

    for p in range(NP):
        fire_idx(p, jnp.int32(p))

    for j, tab in enumerate((onset, rhyme, tone)):
        for k in range(2):
            t0 = jnp.minimum(sid * (2 * STAGE_ROWS), VOCAB - 2 * STAGE_ROWS) \
                + k * STAGE_ROWS
            pltpu.sync_copy(tab.at[pl.ds(t0, STAGE_ROWS)], stg)

            def scale_row(i, _):
                for h in range(D // L):
                    stg[i, pl.ds(h * L, L)] = stg[i, pl.ds(h * L, L)] * SCALE
                return 0

            lax.fori_loop(0, STAGE_ROWS, scale_row, 0)
            pltpu.sync_copy(stg, shared[j].at[pl.ds(t0, STAGE_ROWS)])
    plsc.subcore_barrier()

    for p in range(NP):
        c = jnp.int32(p)
        wait_idx(p, c)
        for j in range(NUM_TABLES):
            fire_gather(j, p, c)

    def quad_body(i, _):
        base = NP * i
        for p in range(NP):
            c_old = base + p
            for j in range(NUM_TABLES):
                wait_gather(j, p, c_old)
            for j in range(NUM_TABLES):
                fire_write(j, p, c_old)
            fire_idx(p, c_old + NP)
        for p in range(NP):
            c_old = base + p
            wait_idx(p, c_old + NP)
            for j in range(NUM_TABLES):
                wait_write(j, p, c_old)
                fire_gather(j, p, c_old + NP)
        return 0

    lax.fori_loop(0, nch // NP - 1, quad_body, 0)

    for p in range(NP):
        c = jnp.int32(nch - NP + p)
        for j in range(NUM_TABLES):
            wait_gather(j, p, c)
        for j in range(NUM_TABLES):
            fire_write(j, p, c)
    for p in range(NP):
        c = jnp.int32(nch - NP + p)
        for j in range(NUM_TABLES):
            wait_write(j, p, c)


def _make(n_batch, seq):
    mesh = plsc.VectorSubcoreMesh(core_axis_name="c", subcore_axis_name="s")
    return pl.kernel(
        _body,
        out_type=jax.ShapeDtypeStruct((seq, n_batch, NUM_TABLES * D),
                                      jnp.float32),
        mesh=mesh,
        compiler_params=pltpu.CompilerParams(needs_layout_passes=False),
        scratch_types=[
            pltpu.VMEM_SHARED((VOCAB, D), jnp.float32),
            pltpu.VMEM_SHARED((VOCAB, D), jnp.float32),
            pltpu.VMEM_SHARED((VOCAB, D), jnp.float32),
            pltpu.VMEM((STAGE_ROWS, D), jnp.float32),
        ] + [pltpu.VMEM((NUM_TABLES, BBLK), jnp.int32)] * NP
          + [pltpu.VMEM((CB, D), jnp.float32)] * (NUM_TABLES * NP)
          + [pltpu.SemaphoreType.DMA] * (2 * NUM_TABLES * NP + NP),
    )


@jax.jit
def kernel(phoneme_tensor, onset_table, rhyme_table, tone_table):
    b, s, _ = phoneme_tensor.shape
    ph_t = phoneme_tensor.astype(jnp.int32).transpose(2, 1, 0)
    out3 = _make(b, s)(ph_t, onset_table, rhyme_table, tone_table)
    return out3.transpose(1, 0, 2)
